# baseline (speedup 1.0000x reference)
import functools

import jax
import jax.numpy as jnp
from jax import lax
from jax.experimental import pallas as pl
from jax.experimental.pallas import tpu as pltpu
from jax.experimental.pallas import tpu_sc as plsc

_NC = 2
_NS = 16
_NW = _NC * _NS
_LANES = 16
_CHUNK = 16384


def kernel(x, pos_emb):
    B, T, C = x.shape
    total = B * T * C
    pe_total = T * C
    per_w = total // _NW
    n_chunks = per_w // _CHUNK

    xf = x.reshape(total)
    pef = pos_emb.reshape(pe_total)

    mesh = plsc.VectorSubcoreMesh(core_axis_name="c", subcore_axis_name="s")

    @functools.partial(
        pl.kernel,
        mesh=mesh,
        out_type=jax.ShapeDtypeStruct((total,), jnp.float32),
        scratch_types=[
            pltpu.VMEM((_CHUNK,), jnp.float32),
            pltpu.VMEM((_CHUNK,), jnp.float32),
        ],
    )
    def sc_add(x_hbm, pe_hbm, out_hbm, xbuf, pebuf):
        wid = lax.axis_index("s") * _NC + lax.axis_index("c")
        x_base = wid * per_w
        pe_base = lax.rem(x_base, pe_total)

        def chunk_body(i, carry):
            xo = x_base + i * _CHUNK
            po = pe_base + i * _CHUNK
            pltpu.sync_copy(x_hbm.at[pl.ds(xo, _CHUNK)], xbuf)
            pltpu.sync_copy(pe_hbm.at[pl.ds(po, _CHUNK)], pebuf)

            def vec_body(j, c):
                s = pl.ds(j * _LANES, _LANES)
                xbuf[s] = xbuf[s] + pebuf[s]
                return c

            lax.fori_loop(0, _CHUNK // _LANES, vec_body, 0)
            pltpu.sync_copy(xbuf, out_hbm.at[pl.ds(xo, _CHUNK)])
            return carry

        lax.fori_loop(0, n_chunks, chunk_body, 0)

    out = sc_add(xf, pef)
    return out.reshape(B, T, C)


# speedup vs baseline: 1.3908x; 1.3908x over previous
import functools

import jax
import jax.numpy as jnp
from jax import lax
from jax.experimental import pallas as pl
from jax.experimental.pallas import tpu as pltpu
from jax.experimental.pallas import tpu_sc as plsc

_NC = 2
_NS = 16
_NW = _NC * _NS
_LANES = 16
_CHUNK = 16384
_UNROLL = 8


def kernel(x, pos_emb):
    B, T, C = x.shape
    total = B * T * C
    pe_total = T * C
    per_w = total // _NW
    n_chunks = per_w // _CHUNK

    xf = x.reshape(total)
    pef = pos_emb.reshape(pe_total)

    mesh = plsc.VectorSubcoreMesh(core_axis_name="c", subcore_axis_name="s")

    @functools.partial(
        pl.kernel,
        mesh=mesh,
        out_type=jax.ShapeDtypeStruct((total,), jnp.float32),
        scratch_types=[
            pltpu.VMEM((_CHUNK,), jnp.float32),
            pltpu.VMEM((_CHUNK,), jnp.float32),
        ],
    )
    def sc_add(x_hbm, pe_hbm, out_hbm, xbuf, pebuf):
        wid = lax.axis_index("s") * _NC + lax.axis_index("c")
        x_base = wid * per_w
        pe_base = lax.rem(x_base, pe_total)

        def chunk_body(i, carry):
            xo = x_base + i * _CHUNK
            po = pe_base + i * _CHUNK
            pltpu.sync_copy(x_hbm.at[pl.ds(xo, _CHUNK)], xbuf)
            pltpu.sync_copy(pe_hbm.at[pl.ds(po, _CHUNK)], pebuf)

            def vec_body(j, c):
                base = j * (_LANES * _UNROLL)
                for k in range(_UNROLL):
                    s = pl.ds(base + k * _LANES, _LANES)
                    xbuf[s] = xbuf[s] + pebuf[s]
                return c

            lax.fori_loop(0, _CHUNK // (_LANES * _UNROLL), vec_body, 0)
            pltpu.sync_copy(xbuf, out_hbm.at[pl.ds(xo, _CHUNK)])
            return carry

        lax.fori_loop(0, n_chunks, chunk_body, 0)

    out = sc_add(xf, pef)
    return out.reshape(B, T, C)


# speedup vs baseline: 8.5480x; 6.1462x over previous
import jax
import jax.numpy as jnp
from jax.experimental import pallas as pl
from jax.experimental.pallas import tpu as pltpu

_ROWS = 2048


def _add_body(x_ref, pe_ref, out_ref):
    out_ref[...] = x_ref[...] + pe_ref[...][None]


def kernel(x, pos_emb):
    B, T, C = x.shape
    grid = (T // _ROWS, B)
    return pl.pallas_call(
        _add_body,
        grid=grid,
        in_specs=[
            pl.BlockSpec((1, _ROWS, C), lambda t, b: (b, t, 0)),
            pl.BlockSpec((_ROWS, C), lambda t, b: (t, 0)),
        ],
        out_specs=pl.BlockSpec((1, _ROWS, C), lambda t, b: (b, t, 0)),
        out_shape=jax.ShapeDtypeStruct((B, T, C), x.dtype),
    )(x, pos_emb)


# speedup vs baseline: 8.5623x; 1.0017x over previous
import jax
from jax.experimental import pallas as pl

_ROWS = 2048


def _add_body(x_ref, pe_ref, out_ref):
    out_ref[...] = x_ref[...] + pe_ref[...][None]


def kernel(x, pos_emb):
    B, T, C = x.shape
    grid = (T // _ROWS, B)
    return pl.pallas_call(
        _add_body,
        grid=grid,
        in_specs=[
            pl.BlockSpec((1, _ROWS, C), lambda t, b: (b, t, 0)),
            pl.BlockSpec((_ROWS, C), lambda t, b: (t, 0)),
        ],
        out_specs=pl.BlockSpec((1, _ROWS, C), lambda t, b: (b, t, 0)),
        out_shape=jax.ShapeDtypeStruct((B, T, C), x.dtype),
    )(x, pos_emb)
